# baseline (speedup 1.0000x reference)
import functools

import jax
import jax.numpy as jnp
from jax import lax
from jax.experimental import pallas as pl
from jax.experimental.pallas import tpu as pltpu
from jax.experimental.pallas import tpu_sc as plsc

B = 8
N = 10000
D = 128
E = 320000
R = 32

NC = 2
NS = 16
NW = NC * NS
EPW = E // NW
K = 80
NCHUNK = EPW // K
HALF = NCHUNK // 2


def _chunk_perms():
    r = lax.broadcasted_iota(jnp.int32, (2 * D, D), 0)
    c = lax.broadcasted_iota(jnp.int32, (2 * D, D), 1)
    feat_a = 32 * (c // 16) + (c % 16)
    pa = (r == feat_a).astype(jnp.float32)
    pb = (r == feat_a + 16).astype(jnp.float32)
    return pa, pb


def _pack_i32(lo_half, hi_half):
    pa, pb = _chunk_perms()
    lo16 = lo_half.astype(jnp.bfloat16)
    hi16 = hi_half.astype(jnp.bfloat16)
    a = jnp.dot(lo16, pa[:D].astype(jnp.bfloat16),
                preferred_element_type=jnp.float32) + \
        jnp.dot(hi16, pa[D:].astype(jnp.bfloat16),
                preferred_element_type=jnp.float32)
    b = jnp.dot(lo16, pb[:D].astype(jnp.bfloat16),
                preferred_element_type=jnp.float32) + \
        jnp.dot(hi16, pb[D:].astype(jnp.bfloat16),
                preferred_element_type=jnp.float32)
    au = lax.bitcast_convert_type(a.astype(jnp.bfloat16), jnp.uint16)
    bu = lax.bitcast_convert_type(b.astype(jnp.bfloat16), jnp.uint16)
    return au.astype(jnp.int32) | (bu.astype(jnp.int32) << 16)


def _node_table_body(a_ref, ws_ref, t_ref):
    a = a_ref[...]
    hs = jnp.dot(a, ws_ref[...], preferred_element_type=jnp.float32)
    t_ref[...] = _pack_i32(a, hs)


def _combo_body(rela_ref, q_ref, wr_ref, wq_ref, b_ref, s_ref):
    rela = rela_ref[...]
    rr = jnp.dot(rela, wr_ref[...], preferred_element_type=jnp.float32)
    qw = jnp.dot(q_ref[...], wq_ref[...], preferred_element_type=jnp.float32)
    qw = qw + b_ref[...]
    rrep = jnp.broadcast_to(rela[:, None, :], (R, B, D)).reshape(R * B, D)
    s2 = (rr[:, None, :] + qw[None, :, :]).reshape(R * B, D)
    s_ref[...] = _pack_i32(rrep, s2)


def _combo_idx_body(rel_ref, bat_ref, out_ref):
    out_ref[...] = rel_ref[...] * 8 + bat_ref[...]


def _edge_body(t_hbm, s_hbm, sub_hbm, c_hbm, w_hbm, out_hbm,
               sub_all, c_all, s_v, hrows0, hrows1, out0, out1,
               w_v, sem_t0, sem_t1, sem_o0, sem_o1):
    wid = lax.axis_index("s") * NC + lax.axis_index("c")
    base = wid * EPW
    pltpu.sync_copy(w_hbm, w_v)
    pltpu.sync_copy(s_hbm, s_v)
    pltpu.sync_copy(sub_hbm.at[pl.ds(base, EPW)], sub_all)
    pltpu.sync_copy(c_hbm.at[pl.ds(base, EPW)], c_all.at[pl.ds(0, EPW)])
    ones16 = jnp.ones((16,), jnp.float32)
    wk = [w_v[pl.ds(k * 16, 16)] for k in range(D // 16)]
    hrows = (hrows0, hrows1)
    outs = (out0, out1)
    sem_t = (sem_t0, sem_t1)
    sem_o = (sem_o0, sem_o1)

    def issue(j, b):
        pltpu.async_copy(t_hbm.at[sub_all.at[pl.ds(j * K, K)]], hrows[b], sem_t[b])

    def wait(j, b):
        pltpu.make_async_copy(
            t_hbm.at[sub_all.at[pl.ds(j * K, K)]], hrows[b], sem_t[b]).wait()

    def compute(j, b):
        hv, ov = hrows[b], outs[b]
        ebase = j * K

        @pl.when(j >= 2)
        def _():
            pltpu.make_async_copy(
                ov, out_hbm.at[pl.ds(base + (j - 2) * K, K)], sem_o[b]).wait()

        def unpack2(ref, row, word_ofs):
            q = ref[row, pl.ds(word_ofs, 16)]
            bc = plsc.bitcast(q, jnp.bfloat16)
            return plsc.unpack(bc, format=plsc.PackFormat.INTERLEAVED)

        @plsc.parallel_loop(0, K, 1, unroll=6)
        def edge_body(e):
            ce = c_all[pl.ds(ebase + e, 16)][0]
            acc = jnp.zeros((16,), jnp.float32)
            for g in range(D // 32):
                hs_a, hs_b = unpack2(hv, e, D // 2 + g * 16)
                cc_a, cc_b = unpack2(s_v, ce, D // 2 + g * 16)
                acc = acc + jnp.maximum(hs_a + cc_a, 0.0) * wk[2 * g]
                acc = acc + jnp.maximum(hs_b + cc_b, 0.0) * wk[2 * g + 1]
            a = jnp.sum(acc)
            alpha = 1.0 / (1.0 + jnp.exp(-a * ones16))
            for g in range(D // 32):
                h_a, h_b = unpack2(hv, e, g * 16)
                r_a, r_b = unpack2(s_v, ce, g * 16)
                ov[e, pl.ds(g * 32, 16)] = h_a * r_a * alpha
                ov[e, pl.ds(g * 32 + 16, 16)] = h_b * r_b * alpha

        pltpu.async_copy(ov, out_hbm.at[pl.ds(base + j * K, K)], sem_o[b])

    issue(jnp.int32(0), 0)

    def chunk_body(i, carry):
        j0 = 2 * i
        j1 = j0 + 1
        issue(j1, 1)
        wait(j0, 0)
        compute(j0, 0)
        issue(j0 + 2, 0)
        wait(j1, 1)
        compute(j1, 1)
        return carry

    lax.fori_loop(0, HALF, chunk_body, jnp.int32(0))
    last = jnp.int32(NCHUNK - 1)
    wait(last, 0)
    compute(last, 0)
    pltpu.make_async_copy(
        out0, out_hbm.at[pl.ds(base + (NCHUNK - 1) * K, K)], sem_o0).wait()
    pltpu.make_async_copy(
        out1, out_hbm.at[pl.ds(base + (NCHUNK - 2) * K, K)], sem_o1).wait()


def kernel(query, q_sub, q_rel, hidden, edges, nodes, rela_embed,
           Ws_attn, Wr_attn, Wqr_attn_W, Wqr_attn_b, W_attn):
    all_ent = hidden.reshape(-1, D)
    blk = 640
    nsteps = all_ent.shape[0] // blk
    erows = E // D
    eblk = erows // nsteps
    node_table = pl.pallas_call(
        _node_table_body,
        grid=(nsteps,),
        in_specs=[
            pl.BlockSpec((blk, D), lambda i: (i, 0)),
            pl.BlockSpec((D, D), lambda i: (0, 0)),
        ],
        out_specs=pl.BlockSpec((blk, D), lambda i: (i, 0)),
        out_shape=jax.ShapeDtypeStruct((all_ent.shape[0], D), jnp.int32),
    )(all_ent, Ws_attn)

    combo_table = pl.pallas_call(
        _combo_body,
        out_shape=jax.ShapeDtypeStruct((R * B, D), jnp.int32),
    )(rela_embed, query, Wr_attn, Wqr_attn_W, Wqr_attn_b.reshape(1, D))

    rel2d = edges[:, 2].reshape(erows, D)
    bat2d = edges[:, 0].reshape(erows, D)
    combo_idx = pl.pallas_call(
        _combo_idx_body,
        out_shape=jax.ShapeDtypeStruct((erows, D), jnp.int32),
    )(rel2d, bat2d).reshape(E)
    sub_idx = edges[:, 1]

    mesh = plsc.VectorSubcoreMesh(
        core_axis_name="c", subcore_axis_name="s",
        num_cores=NC, num_subcores=NS)
    sc = functools.partial(
        pl.kernel,
        mesh=mesh,
        compiler_params=pltpu.CompilerParams(needs_layout_passes=False),
        out_type=jax.ShapeDtypeStruct((E, D), jnp.float32),
        scratch_types=[
            pltpu.VMEM((EPW,), jnp.int32),
            pltpu.VMEM((EPW + 16,), jnp.int32),
            pltpu.VMEM((R * B, D), jnp.int32),
            pltpu.VMEM((K, D), jnp.int32),
            pltpu.VMEM((K, D), jnp.int32),
            pltpu.VMEM((K, D), jnp.float32),
            pltpu.VMEM((K, D), jnp.float32),
            pltpu.VMEM((D,), jnp.float32),
            pltpu.SemaphoreType.DMA,
            pltpu.SemaphoreType.DMA,
            pltpu.SemaphoreType.DMA,
            pltpu.SemaphoreType.DMA,
        ],
    )(_edge_body)
    return sc(node_table, combo_table, sub_idx, combo_idx, W_attn.reshape(D))


# speedup vs baseline: 1.2884x; 1.2884x over previous
import functools

import jax
import jax.numpy as jnp
from jax import lax
from jax.experimental import pallas as pl
from jax.experimental.pallas import tpu as pltpu
from jax.experimental.pallas import tpu_sc as plsc

B = 8
N = 10000
D = 128
E = 320000
R = 32

NC = 2
NS = 16
NW = NC * NS
EPW = E // NW
K = 80
NCHUNK = EPW // K
HALF = NCHUNK // 2


def _chunk_perms():
    r = lax.broadcasted_iota(jnp.int32, (2 * D, D), 0)
    c = lax.broadcasted_iota(jnp.int32, (2 * D, D), 1)
    feat_a = 32 * (c // 16) + (c % 16)
    pa = (r == feat_a).astype(jnp.float32)
    pb = (r == feat_a + 16).astype(jnp.float32)
    return pa, pb


def _pack_i32(lo_half, hi_half):
    pa, pb = _chunk_perms()
    lo16 = lo_half.astype(jnp.bfloat16)
    hi16 = hi_half.astype(jnp.bfloat16)
    a = jnp.dot(lo16, pa[:D].astype(jnp.bfloat16),
                preferred_element_type=jnp.float32) + \
        jnp.dot(hi16, pa[D:].astype(jnp.bfloat16),
                preferred_element_type=jnp.float32)
    b = jnp.dot(lo16, pb[:D].astype(jnp.bfloat16),
                preferred_element_type=jnp.float32) + \
        jnp.dot(hi16, pb[D:].astype(jnp.bfloat16),
                preferred_element_type=jnp.float32)
    au = lax.bitcast_convert_type(a.astype(jnp.bfloat16), jnp.uint16)
    bu = lax.bitcast_convert_type(b.astype(jnp.bfloat16), jnp.uint16)
    return au.astype(jnp.int32) | (bu.astype(jnp.int32) << 16)


def _node_table_body(a_ref, ws_ref, t_ref):
    a = a_ref[...]
    hs = jnp.dot(a, ws_ref[...], preferred_element_type=jnp.float32)
    t_ref[...] = _pack_i32(a, hs)


def _combo_body(rela_ref, q_ref, wr_ref, wq_ref, b_ref, s_ref):
    rela = rela_ref[...]
    rr = jnp.dot(rela, wr_ref[...], preferred_element_type=jnp.float32)
    qw = jnp.dot(q_ref[...], wq_ref[...], preferred_element_type=jnp.float32)
    qw = qw + b_ref[...]
    rrep = jnp.broadcast_to(rela[:, None, :], (R, B, D)).reshape(R * B, D)
    s2 = (rr[:, None, :] + qw[None, :, :]).reshape(R * B, D)
    s_ref[...] = _pack_i32(rrep, s2)


def _combo_idx_body(rel_ref, bat_ref, out_ref):
    out_ref[...] = rel_ref[...] * 8 + bat_ref[...]


def _edge_body(t_hbm, s_hbm, sub_hbm, c_hbm, w_hbm, out_hbm,
               sub_all, c_all, s_v, hrows0, hrows1, out0, out1,
               w_v, sem_t0, sem_t1, sem_o0, sem_o1):
    wid = lax.axis_index("s") * NC + lax.axis_index("c")
    base = wid * EPW
    pltpu.sync_copy(w_hbm, w_v)
    pltpu.sync_copy(s_hbm, s_v)
    pltpu.sync_copy(sub_hbm.at[pl.ds(base, EPW)], sub_all)
    pltpu.sync_copy(c_hbm.at[pl.ds(base, EPW)], c_all.at[pl.ds(0, EPW)])
    ones16 = jnp.ones((16,), jnp.float32)
    wk = [w_v[pl.ds(k * 16, 16)] for k in range(D // 16)]
    hrows = (hrows0, hrows1)
    outs = (out0, out1)
    sem_t = (sem_t0, sem_t1)
    sem_o = (sem_o0, sem_o1)

    def issue(j, b):
        pltpu.async_copy(t_hbm.at[sub_all.at[pl.ds(j * K, K)]], hrows[b], sem_t[b])

    def wait(j, b):
        pltpu.make_async_copy(
            t_hbm.at[sub_all.at[pl.ds(j * K, K)]], hrows[b], sem_t[b]).wait()

    def compute(j, b):
        hv, ov = hrows[b], outs[b]
        ebase = j * K

        @pl.when(j >= 2)
        def _():
            pltpu.make_async_copy(
                ov, out_hbm.at[pl.ds(base + (j - 2) * K, K)], sem_o[b]).wait()

        def unpack2(ref, row, word_ofs):
            q = ref[row, pl.ds(word_ofs, 16)]
            bc = plsc.bitcast(q, jnp.bfloat16)
            return plsc.unpack(bc, format=plsc.PackFormat.INTERLEAVED)

        @plsc.parallel_loop(0, K, 1, unroll=4)
        def edge_body(e):
            ce = c_all[pl.ds(ebase + e, 16)][0]
            acc = jnp.zeros((16,), jnp.float32)
            for g in range(D // 32):
                hs_a, hs_b = unpack2(hv, e, D // 2 + g * 16)
                cc_a, cc_b = unpack2(s_v, ce, D // 2 + g * 16)
                acc = acc + jnp.maximum(hs_a + cc_a, 0.0) * wk[2 * g]
                acc = acc + jnp.maximum(hs_b + cc_b, 0.0) * wk[2 * g + 1]
            a = jnp.sum(acc)
            alpha = 1.0 / (1.0 + jnp.exp(-a * ones16))
            for g in range(D // 32):
                h_a, h_b = unpack2(hv, e, g * 16)
                r_a, r_b = unpack2(s_v, ce, g * 16)
                ov[e, pl.ds(g * 32, 16)] = h_a * r_a * alpha
                ov[e, pl.ds(g * 32 + 16, 16)] = h_b * r_b * alpha

        pltpu.async_copy(ov, out_hbm.at[pl.ds(base + j * K, K)], sem_o[b])

    issue(jnp.int32(0), 0)

    def chunk_body(i, carry):
        j0 = 2 * i
        j1 = j0 + 1
        issue(j1, 1)
        wait(j0, 0)
        compute(j0, 0)
        issue(j0 + 2, 0)
        wait(j1, 1)
        compute(j1, 1)
        return carry

    lax.fori_loop(0, HALF, chunk_body, jnp.int32(0))
    last = jnp.int32(NCHUNK - 1)
    wait(last, 0)
    compute(last, 0)
    pltpu.make_async_copy(
        out0, out_hbm.at[pl.ds(base + (NCHUNK - 1) * K, K)], sem_o0).wait()
    pltpu.make_async_copy(
        out1, out_hbm.at[pl.ds(base + (NCHUNK - 2) * K, K)], sem_o1).wait()


def kernel(query, q_sub, q_rel, hidden, edges, nodes, rela_embed,
           Ws_attn, Wr_attn, Wqr_attn_W, Wqr_attn_b, W_attn):
    all_ent = hidden.reshape(-1, D)
    blk = 640
    nsteps = all_ent.shape[0] // blk
    erows = E // D
    eblk = erows // nsteps
    node_table = pl.pallas_call(
        _node_table_body,
        grid=(nsteps,),
        in_specs=[
            pl.BlockSpec((blk, D), lambda i: (i, 0)),
            pl.BlockSpec((D, D), lambda i: (0, 0)),
        ],
        out_specs=pl.BlockSpec((blk, D), lambda i: (i, 0)),
        out_shape=jax.ShapeDtypeStruct((all_ent.shape[0], D), jnp.int32),
    )(all_ent, Ws_attn)

    combo_table = pl.pallas_call(
        _combo_body,
        out_shape=jax.ShapeDtypeStruct((R * B, D), jnp.int32),
    )(rela_embed, query, Wr_attn, Wqr_attn_W, Wqr_attn_b.reshape(1, D))

    rel2d = edges[:, 2].reshape(erows, D)
    bat2d = edges[:, 0].reshape(erows, D)
    combo_idx = pl.pallas_call(
        _combo_idx_body,
        out_shape=jax.ShapeDtypeStruct((erows, D), jnp.int32),
    )(rel2d, bat2d).reshape(E)
    sub_idx = edges[:, 1]

    mesh = plsc.VectorSubcoreMesh(
        core_axis_name="c", subcore_axis_name="s",
        num_cores=NC, num_subcores=NS)
    sc = functools.partial(
        pl.kernel,
        mesh=mesh,
        compiler_params=pltpu.CompilerParams(needs_layout_passes=False),
        out_type=jax.ShapeDtypeStruct((E, D), jnp.float32),
        scratch_types=[
            pltpu.VMEM((EPW,), jnp.int32),
            pltpu.VMEM((EPW + 16,), jnp.int32),
            pltpu.VMEM((R * B, D), jnp.int32),
            pltpu.VMEM((K, D), jnp.int32),
            pltpu.VMEM((K, D), jnp.int32),
            pltpu.VMEM((K, D), jnp.float32),
            pltpu.VMEM((K, D), jnp.float32),
            pltpu.VMEM((D,), jnp.float32),
            pltpu.SemaphoreType.DMA,
            pltpu.SemaphoreType.DMA,
            pltpu.SemaphoreType.DMA,
            pltpu.SemaphoreType.DMA,
        ],
    )(_edge_body)
    return sc(node_table, combo_table, sub_idx, combo_idx, W_attn.reshape(D))


# speedup vs baseline: 1.2930x; 1.0036x over previous
import functools

import jax
import jax.numpy as jnp
from jax import lax
from jax.experimental import pallas as pl
from jax.experimental.pallas import tpu as pltpu
from jax.experimental.pallas import tpu_sc as plsc

B = 8
N = 10000
D = 128
E = 320000
R = 32

NC = 2
NS = 16
NW = NC * NS
EPW = E // NW
K = 80
NCHUNK = EPW // K
HALF = NCHUNK // 2


def _chunk_perms():
    r = lax.broadcasted_iota(jnp.int32, (2 * D, D), 0)
    c = lax.broadcasted_iota(jnp.int32, (2 * D, D), 1)
    feat_a = 32 * (c // 16) + (c % 16)
    pa = (r == feat_a).astype(jnp.float32)
    pb = (r == feat_a + 16).astype(jnp.float32)
    return pa, pb


def _pack_i32(lo_half, hi_half):
    pa, pb = _chunk_perms()
    lo16 = lo_half.astype(jnp.bfloat16)
    hi16 = hi_half.astype(jnp.bfloat16)
    a = jnp.dot(lo16, pa[:D].astype(jnp.bfloat16),
                preferred_element_type=jnp.float32) + \
        jnp.dot(hi16, pa[D:].astype(jnp.bfloat16),
                preferred_element_type=jnp.float32)
    b = jnp.dot(lo16, pb[:D].astype(jnp.bfloat16),
                preferred_element_type=jnp.float32) + \
        jnp.dot(hi16, pb[D:].astype(jnp.bfloat16),
                preferred_element_type=jnp.float32)
    au = lax.bitcast_convert_type(a.astype(jnp.bfloat16), jnp.uint16)
    bu = lax.bitcast_convert_type(b.astype(jnp.bfloat16), jnp.uint16)
    return au.astype(jnp.int32) | (bu.astype(jnp.int32) << 16)


def _prep_body(a_ref, ws_ref, rela_ref, q_ref, wr_ref, wq_ref, b_ref,
               t_ref, s_ref):
    a = a_ref[...]
    hs = jnp.dot(a, ws_ref[...], preferred_element_type=jnp.float32)
    t_ref[...] = _pack_i32(a, hs)

    @pl.when(pl.program_id(0) == 0)
    def _():
        rela = rela_ref[...]
        rr = jnp.dot(rela, wr_ref[...], preferred_element_type=jnp.float32)
        qw = jnp.dot(q_ref[...], wq_ref[...], preferred_element_type=jnp.float32)
        qw = qw + b_ref[...]
        rrep = jnp.broadcast_to(rela[:, None, :], (R, B, D)).reshape(R * B, D)
        s2 = (rr[:, None, :] + qw[None, :, :]).reshape(R * B, D)
        s_ref[...] = _pack_i32(rrep, s2)


def _edge_body(t_hbm, s_hbm, sub_hbm, rel_hbm, bat_hbm, w_hbm, out_hbm,
               sub_all, c_all, bat_all, s_v, hrows0, hrows1, out0, out1,
               w_v, sem_t0, sem_t1, sem_o0, sem_o1):
    wid = lax.axis_index("s") * NC + lax.axis_index("c")
    base = wid * EPW
    pltpu.sync_copy(w_hbm, w_v)
    pltpu.sync_copy(s_hbm, s_v)
    pltpu.sync_copy(sub_hbm.at[pl.ds(base, EPW)], sub_all)
    pltpu.sync_copy(rel_hbm.at[pl.ds(base, EPW)], c_all.at[pl.ds(0, EPW)])
    pltpu.sync_copy(bat_hbm.at[pl.ds(base, EPW)], bat_all)

    @plsc.parallel_loop(0, EPW, 16, unroll=4)
    def idx_body(i):
        sl = pl.ds(i, 16)
        c_all[sl] = c_all[sl] * 8 + bat_all[sl]

    ones16 = jnp.ones((16,), jnp.float32)
    wk = [w_v[pl.ds(k * 16, 16)] for k in range(D // 16)]
    hrows = (hrows0, hrows1)
    outs = (out0, out1)
    sem_t = (sem_t0, sem_t1)
    sem_o = (sem_o0, sem_o1)

    def issue(j, b):
        pltpu.async_copy(t_hbm.at[sub_all.at[pl.ds(j * K, K)]], hrows[b], sem_t[b])

    def wait(j, b):
        pltpu.make_async_copy(
            t_hbm.at[sub_all.at[pl.ds(j * K, K)]], hrows[b], sem_t[b]).wait()

    def compute(j, b):
        hv, ov = hrows[b], outs[b]
        ebase = j * K

        @pl.when(j >= 2)
        def _():
            pltpu.make_async_copy(
                ov, out_hbm.at[pl.ds(base + (j - 2) * K, K)], sem_o[b]).wait()

        def unpack2(ref, row, word_ofs):
            q = ref[row, pl.ds(word_ofs, 16)]
            bc = plsc.bitcast(q, jnp.bfloat16)
            return plsc.unpack(bc, format=plsc.PackFormat.INTERLEAVED)

        @plsc.parallel_loop(0, K, 1, unroll=4)
        def edge_body(e):
            ce = c_all[pl.ds(ebase + e, 16)][0]
            acc = jnp.zeros((16,), jnp.float32)
            for g in range(D // 32):
                hs_a, hs_b = unpack2(hv, e, D // 2 + g * 16)
                cc_a, cc_b = unpack2(s_v, ce, D // 2 + g * 16)
                acc = acc + jnp.maximum(hs_a + cc_a, 0.0) * wk[2 * g]
                acc = acc + jnp.maximum(hs_b + cc_b, 0.0) * wk[2 * g + 1]
            a = jnp.sum(acc)
            alpha = 1.0 / (1.0 + jnp.exp(-a * ones16))
            for g in range(D // 32):
                h_a, h_b = unpack2(hv, e, g * 16)
                r_a, r_b = unpack2(s_v, ce, g * 16)
                ov[e, pl.ds(g * 32, 16)] = h_a * r_a * alpha
                ov[e, pl.ds(g * 32 + 16, 16)] = h_b * r_b * alpha

        pltpu.async_copy(ov, out_hbm.at[pl.ds(base + j * K, K)], sem_o[b])

    issue(jnp.int32(0), 0)

    def chunk_body(i, carry):
        j0 = 2 * i
        j1 = j0 + 1
        issue(j1, 1)
        wait(j0, 0)
        compute(j0, 0)
        issue(j0 + 2, 0)
        wait(j1, 1)
        compute(j1, 1)
        return carry

    lax.fori_loop(0, HALF, chunk_body, jnp.int32(0))
    last = jnp.int32(NCHUNK - 1)
    wait(last, 0)
    compute(last, 0)
    pltpu.make_async_copy(
        out0, out_hbm.at[pl.ds(base + (NCHUNK - 1) * K, K)], sem_o0).wait()
    pltpu.make_async_copy(
        out1, out_hbm.at[pl.ds(base + (NCHUNK - 2) * K, K)], sem_o1).wait()


def kernel(query, q_sub, q_rel, hidden, edges, nodes, rela_embed,
           Ws_attn, Wr_attn, Wqr_attn_W, Wqr_attn_b, W_attn):
    all_ent = hidden.reshape(-1, D)
    blk = 640
    nsteps = all_ent.shape[0] // blk
    erows = E // D
    eblk = erows // nsteps
    zero_map = lambda i: (0, 0)
    node_table, combo_table = pl.pallas_call(
        _prep_body,
        grid=(nsteps,),
        in_specs=[
            pl.BlockSpec((blk, D), lambda i: (i, 0)),
            pl.BlockSpec((D, D), zero_map),
            pl.BlockSpec((R, D), zero_map),
            pl.BlockSpec((B, D), zero_map),
            pl.BlockSpec((D, D), zero_map),
            pl.BlockSpec((D, D), zero_map),
            pl.BlockSpec((1, D), zero_map),
        ],
        out_specs=[
            pl.BlockSpec((blk, D), lambda i: (i, 0)),
            pl.BlockSpec((R * B, D), zero_map),
        ],
        out_shape=[
            jax.ShapeDtypeStruct((all_ent.shape[0], D), jnp.int32),
            jax.ShapeDtypeStruct((R * B, D), jnp.int32),
        ],
    )(all_ent, Ws_attn, rela_embed, query, Wr_attn, Wqr_attn_W,
      Wqr_attn_b.reshape(1, D))

    sub_idx = edges[:, 1]
    rel_idx = edges[:, 2]
    bat_idx = edges[:, 0]

    mesh = plsc.VectorSubcoreMesh(
        core_axis_name="c", subcore_axis_name="s",
        num_cores=NC, num_subcores=NS)
    sc = functools.partial(
        pl.kernel,
        mesh=mesh,
        compiler_params=pltpu.CompilerParams(needs_layout_passes=False),
        out_type=jax.ShapeDtypeStruct((E, D), jnp.float32),
        scratch_types=[
            pltpu.VMEM((EPW,), jnp.int32),
            pltpu.VMEM((EPW + 16,), jnp.int32),
            pltpu.VMEM((EPW,), jnp.int32),
            pltpu.VMEM((R * B, D), jnp.int32),
            pltpu.VMEM((K, D), jnp.int32),
            pltpu.VMEM((K, D), jnp.int32),
            pltpu.VMEM((K, D), jnp.float32),
            pltpu.VMEM((K, D), jnp.float32),
            pltpu.VMEM((D,), jnp.float32),
            pltpu.SemaphoreType.DMA,
            pltpu.SemaphoreType.DMA,
            pltpu.SemaphoreType.DMA,
            pltpu.SemaphoreType.DMA,
        ],
    )(_edge_body)
    return sc(node_table, combo_table, sub_idx, rel_idx, bat_idx,
              W_attn.reshape(D))


# speedup vs baseline: 1.3080x; 1.0116x over previous
import functools

import jax
import jax.numpy as jnp
from jax import lax
from jax.experimental import pallas as pl
from jax.experimental.pallas import tpu as pltpu
from jax.experimental.pallas import tpu_sc as plsc

B = 8
N = 10000
D = 128
E = 320000
R = 32

NC = 2
NS = 16
NW = NC * NS
EPW = E // NW
K = 80
NCHUNK = EPW // K
HALF = NCHUNK // 2


def _chunk_perms():
    r = lax.broadcasted_iota(jnp.int32, (2 * D, D), 0)
    c = lax.broadcasted_iota(jnp.int32, (2 * D, D), 1)
    feat_a = 32 * (c // 16) + (c % 16)
    pa = (r == feat_a).astype(jnp.float32)
    pb = (r == feat_a + 16).astype(jnp.float32)
    return pa, pb


def _pack_i32(lo_half, hi_half):
    pa, pb = _chunk_perms()
    lo16 = lo_half.astype(jnp.bfloat16)
    hi16 = hi_half.astype(jnp.bfloat16)
    a = jnp.dot(lo16, pa[:D].astype(jnp.bfloat16),
                preferred_element_type=jnp.float32) + \
        jnp.dot(hi16, pa[D:].astype(jnp.bfloat16),
                preferred_element_type=jnp.float32)
    b = jnp.dot(lo16, pb[:D].astype(jnp.bfloat16),
                preferred_element_type=jnp.float32) + \
        jnp.dot(hi16, pb[D:].astype(jnp.bfloat16),
                preferred_element_type=jnp.float32)
    au = lax.bitcast_convert_type(a.astype(jnp.bfloat16), jnp.uint16)
    bu = lax.bitcast_convert_type(b.astype(jnp.bfloat16), jnp.uint16)
    return au.astype(jnp.int32) | (bu.astype(jnp.int32) << 16)


def _prep_body(a_ref, ws_ref, rela_ref, q_ref, wr_ref, wq_ref, b_ref,
               t_ref, s_ref):
    pa, pb = _chunk_perms()
    ws = ws_ref[...]
    ma = (pa[:D] + jnp.dot(ws, pa[D:], preferred_element_type=jnp.float32))
    mb = (pb[:D] + jnp.dot(ws, pb[D:], preferred_element_type=jnp.float32))
    a16 = a_ref[...].astype(jnp.bfloat16)
    va = jnp.dot(a16, ma.astype(jnp.bfloat16), preferred_element_type=jnp.float32)
    vb = jnp.dot(a16, mb.astype(jnp.bfloat16), preferred_element_type=jnp.float32)
    au = lax.bitcast_convert_type(va.astype(jnp.bfloat16), jnp.uint16)
    bu = lax.bitcast_convert_type(vb.astype(jnp.bfloat16), jnp.uint16)
    t_ref[...] = au.astype(jnp.int32) | (bu.astype(jnp.int32) << 16)

    @pl.when(pl.program_id(0) == 0)
    def _():
        rela = rela_ref[...]
        rr = jnp.dot(rela, wr_ref[...], preferred_element_type=jnp.float32)
        qw = jnp.dot(q_ref[...], wq_ref[...], preferred_element_type=jnp.float32)
        qw = qw + b_ref[...]
        rrep = jnp.broadcast_to(rela[:, None, :], (R, B, D)).reshape(R * B, D)
        s2 = (rr[:, None, :] + qw[None, :, :]).reshape(R * B, D)
        s_ref[...] = _pack_i32(rrep, s2)


def _edge_body(t_hbm, s_hbm, sub_hbm, rel_hbm, bat_hbm, w_hbm, out_hbm,
               sub_all, c_all, bat_all, s_v, hrows0, hrows1, out0, out1,
               w_v, sem_t0, sem_t1, sem_o0, sem_o1):
    wid = lax.axis_index("s") * NC + lax.axis_index("c")
    base = wid * EPW
    pltpu.sync_copy(w_hbm, w_v)
    pltpu.sync_copy(s_hbm, s_v)
    pltpu.sync_copy(sub_hbm.at[pl.ds(base, EPW)], sub_all)
    pltpu.sync_copy(rel_hbm.at[pl.ds(base, EPW)], c_all.at[pl.ds(0, EPW)])
    pltpu.sync_copy(bat_hbm.at[pl.ds(base, EPW)], bat_all)

    @plsc.parallel_loop(0, EPW, 16, unroll=4)
    def idx_body(i):
        sl = pl.ds(i, 16)
        c_all[sl] = c_all[sl] * 8 + bat_all[sl]

    ones16 = jnp.ones((16,), jnp.float32)
    wk = [w_v[pl.ds(k * 16, 16)] for k in range(D // 16)]
    hrows = (hrows0, hrows1)
    outs = (out0, out1)
    sem_t = (sem_t0, sem_t1)
    sem_o = (sem_o0, sem_o1)

    def issue(j, b):
        pltpu.async_copy(t_hbm.at[sub_all.at[pl.ds(j * K, K)]], hrows[b], sem_t[b])

    def wait(j, b):
        pltpu.make_async_copy(
            t_hbm.at[sub_all.at[pl.ds(j * K, K)]], hrows[b], sem_t[b]).wait()

    def compute(j, b):
        hv, ov = hrows[b], outs[b]
        ebase = j * K

        @pl.when(j >= 2)
        def _():
            pltpu.make_async_copy(
                ov, out_hbm.at[pl.ds(base + (j - 2) * K, K)], sem_o[b]).wait()

        def unpack2(ref, row, word_ofs):
            q = ref[row, pl.ds(word_ofs, 16)]
            bc = plsc.bitcast(q, jnp.bfloat16)
            return plsc.unpack(bc, format=plsc.PackFormat.INTERLEAVED)

        @plsc.parallel_loop(0, K, 1, unroll=4)
        def edge_body(e):
            ce = c_all[pl.ds(ebase + e, 16)][0]
            acc = jnp.zeros((16,), jnp.float32)
            for g in range(D // 32):
                hs_a, hs_b = unpack2(hv, e, D // 2 + g * 16)
                cc_a, cc_b = unpack2(s_v, ce, D // 2 + g * 16)
                acc = acc + jnp.maximum(hs_a + cc_a, 0.0) * wk[2 * g]
                acc = acc + jnp.maximum(hs_b + cc_b, 0.0) * wk[2 * g + 1]
            a = jnp.sum(acc)
            alpha = 1.0 / (1.0 + jnp.exp(-a * ones16))
            for g in range(D // 32):
                h_a, h_b = unpack2(hv, e, g * 16)
                r_a, r_b = unpack2(s_v, ce, g * 16)
                ov[e, pl.ds(g * 32, 16)] = h_a * r_a * alpha
                ov[e, pl.ds(g * 32 + 16, 16)] = h_b * r_b * alpha

        pltpu.async_copy(ov, out_hbm.at[pl.ds(base + j * K, K)], sem_o[b])

    issue(jnp.int32(0), 0)

    def chunk_body(i, carry):
        j0 = 2 * i
        j1 = j0 + 1
        issue(j1, 1)
        wait(j0, 0)
        compute(j0, 0)
        issue(j0 + 2, 0)
        wait(j1, 1)
        compute(j1, 1)
        return carry

    lax.fori_loop(0, HALF, chunk_body, jnp.int32(0))
    last = jnp.int32(NCHUNK - 1)
    wait(last, 0)
    compute(last, 0)
    pltpu.make_async_copy(
        out0, out_hbm.at[pl.ds(base + (NCHUNK - 1) * K, K)], sem_o0).wait()
    pltpu.make_async_copy(
        out1, out_hbm.at[pl.ds(base + (NCHUNK - 2) * K, K)], sem_o1).wait()


def kernel(query, q_sub, q_rel, hidden, edges, nodes, rela_embed,
           Ws_attn, Wr_attn, Wqr_attn_W, Wqr_attn_b, W_attn):
    all_ent = hidden.reshape(-1, D)
    blk = 640
    nsteps = all_ent.shape[0] // blk
    erows = E // D
    eblk = erows // nsteps
    zero_map = lambda i: (0, 0)
    node_table, combo_table = pl.pallas_call(
        _prep_body,
        grid=(nsteps,),
        in_specs=[
            pl.BlockSpec((blk, D), lambda i: (i, 0)),
            pl.BlockSpec((D, D), zero_map),
            pl.BlockSpec((R, D), zero_map),
            pl.BlockSpec((B, D), zero_map),
            pl.BlockSpec((D, D), zero_map),
            pl.BlockSpec((D, D), zero_map),
            pl.BlockSpec((1, D), zero_map),
        ],
        out_specs=[
            pl.BlockSpec((blk, D), lambda i: (i, 0)),
            pl.BlockSpec((R * B, D), zero_map),
        ],
        out_shape=[
            jax.ShapeDtypeStruct((all_ent.shape[0], D), jnp.int32),
            jax.ShapeDtypeStruct((R * B, D), jnp.int32),
        ],
    )(all_ent, Ws_attn, rela_embed, query, Wr_attn, Wqr_attn_W,
      Wqr_attn_b.reshape(1, D))

    sub_idx = edges[:, 1]
    rel_idx = edges[:, 2]
    bat_idx = edges[:, 0]

    mesh = plsc.VectorSubcoreMesh(
        core_axis_name="c", subcore_axis_name="s",
        num_cores=NC, num_subcores=NS)
    sc = functools.partial(
        pl.kernel,
        mesh=mesh,
        compiler_params=pltpu.CompilerParams(needs_layout_passes=False),
        out_type=jax.ShapeDtypeStruct((E, D), jnp.float32),
        scratch_types=[
            pltpu.VMEM((EPW,), jnp.int32),
            pltpu.VMEM((EPW + 16,), jnp.int32),
            pltpu.VMEM((EPW,), jnp.int32),
            pltpu.VMEM((R * B, D), jnp.int32),
            pltpu.VMEM((K, D), jnp.int32),
            pltpu.VMEM((K, D), jnp.int32),
            pltpu.VMEM((K, D), jnp.float32),
            pltpu.VMEM((K, D), jnp.float32),
            pltpu.VMEM((D,), jnp.float32),
            pltpu.SemaphoreType.DMA,
            pltpu.SemaphoreType.DMA,
            pltpu.SemaphoreType.DMA,
            pltpu.SemaphoreType.DMA,
        ],
    )(_edge_body)
    return sc(node_table, combo_table, sub_idx, rel_idx, bat_idx,
              W_attn.reshape(D))


# speedup vs baseline: 1.5229x; 1.1643x over previous
import functools

import jax
import jax.numpy as jnp
from jax import lax
from jax.experimental import pallas as pl
from jax.experimental.pallas import tpu as pltpu
from jax.experimental.pallas import tpu_sc as plsc

B = 8
N = 10000
D = 128
E = 320000
R = 32

NC = 2
NS = 16
NW = NC * NS
EPW = E // NW
K = 80
NCHUNK = EPW // K
HALF = NCHUNK // 2


def _chunk_perms():
    r = lax.broadcasted_iota(jnp.int32, (2 * D, D), 0)
    c = lax.broadcasted_iota(jnp.int32, (2 * D, D), 1)
    feat_a = 32 * (c // 16) + (c % 16)
    pa = (r == feat_a).astype(jnp.float32)
    pb = (r == feat_a + 16).astype(jnp.float32)
    return pa, pb


def _pack_i32(lo_half, hi_half):
    pa, pb = _chunk_perms()
    lo16 = lo_half.astype(jnp.bfloat16)
    hi16 = hi_half.astype(jnp.bfloat16)
    a = jnp.dot(lo16, pa[:D].astype(jnp.bfloat16),
                preferred_element_type=jnp.float32) + \
        jnp.dot(hi16, pa[D:].astype(jnp.bfloat16),
                preferred_element_type=jnp.float32)
    b = jnp.dot(lo16, pb[:D].astype(jnp.bfloat16),
                preferred_element_type=jnp.float32) + \
        jnp.dot(hi16, pb[D:].astype(jnp.bfloat16),
                preferred_element_type=jnp.float32)
    au = lax.bitcast_convert_type(a.astype(jnp.bfloat16), jnp.uint16)
    bu = lax.bitcast_convert_type(b.astype(jnp.bfloat16), jnp.uint16)
    return au.astype(jnp.int32) | (bu.astype(jnp.int32) << 16)


def _prep_body(a_ref, ws_ref, rela_ref, q_ref, wr_ref, wq_ref, b_ref,
               t_ref, s_ref):
    pa, pb = _chunk_perms()
    ws = ws_ref[...]
    ma = (pa[:D] + jnp.dot(ws, pa[D:], preferred_element_type=jnp.float32))
    mb = (pb[:D] + jnp.dot(ws, pb[D:], preferred_element_type=jnp.float32))
    a16 = a_ref[...].astype(jnp.bfloat16)
    va = jnp.dot(a16, ma.astype(jnp.bfloat16), preferred_element_type=jnp.float32)
    vb = jnp.dot(a16, mb.astype(jnp.bfloat16), preferred_element_type=jnp.float32)
    au = lax.bitcast_convert_type(va.astype(jnp.bfloat16), jnp.uint16)
    bu = lax.bitcast_convert_type(vb.astype(jnp.bfloat16), jnp.uint16)
    t_ref[...] = au.astype(jnp.int32) | (bu.astype(jnp.int32) << 16)

    @pl.when(pl.program_id(0) == 0)
    def _():
        rela = rela_ref[...]
        rr = jnp.dot(rela, wr_ref[...], preferred_element_type=jnp.float32)
        qw = jnp.dot(q_ref[...], wq_ref[...], preferred_element_type=jnp.float32)
        qw = qw + b_ref[...]
        rrep = jnp.broadcast_to(rela[:, None, :], (R, B, D)).reshape(R * B, D)
        s2 = (rr[:, None, :] + qw[None, :, :]).reshape(R * B, D)
        s_ref[...] = _pack_i32(rrep, s2)


def _edge_body(t_hbm, s_hbm, sub_hbm, rel_hbm, bat_hbm, w_hbm, out_hbm,
               sub_all, c_all, bat_all, s_v, hrows0, hrows1, out0, out1,
               w_v, sem_t0, sem_t1, sem_o0, sem_o1):
    wid = lax.axis_index("s") * NC + lax.axis_index("c")
    base = wid * EPW
    pltpu.sync_copy(w_hbm, w_v)
    pltpu.sync_copy(s_hbm, s_v)
    pltpu.sync_copy(sub_hbm.at[pl.ds(base, EPW)], sub_all)
    pltpu.sync_copy(rel_hbm.at[pl.ds(base, EPW)], c_all.at[pl.ds(0, EPW)])
    pltpu.sync_copy(bat_hbm.at[pl.ds(base, EPW)], bat_all)

    @plsc.parallel_loop(0, EPW, 16, unroll=4)
    def idx_body(i):
        sl = pl.ds(i, 16)
        c_all[sl] = c_all[sl] * 8 + bat_all[sl]

    ones16 = jnp.ones((16,), jnp.float32)
    wk = [w_v[pl.ds(k * 16, 16)] for k in range(D // 16)]
    hrows = (hrows0, hrows1)
    outs = (out0, out1)
    sem_t = (sem_t0, sem_t1)
    sem_o = (sem_o0, sem_o1)

    def issue(j, b):
        pltpu.async_copy(t_hbm.at[sub_all.at[pl.ds(j * K, K)]], hrows[b], sem_t[b])

    def wait(j, b):
        pltpu.make_async_copy(
            t_hbm.at[sub_all.at[pl.ds(j * K, K)]], hrows[b], sem_t[b]).wait()

    def compute(j, b):
        hv, ov = hrows[b], outs[b]
        ebase = j * K

        @pl.when(j >= 2)
        def _():
            pltpu.make_async_copy(
                ov, out_hbm.at[pl.ds(base + (j - 2) * K, K)], sem_o[b]).wait()

        def unpack2(ref, row, word_ofs):
            q = ref[row, pl.ds(word_ofs, 16)]
            bc = plsc.bitcast(q, jnp.bfloat16)
            return plsc.unpack(bc, format=plsc.PackFormat.INTERLEAVED)

        @plsc.parallel_loop(0, K, 1, unroll=4)
        def edge_body(e):
            ce = c_all[pl.ds(ebase + e, 16)][0]
            acc = jnp.zeros((16,), jnp.float32)
            for g in range(D // 32):
                hs_a, hs_b = unpack2(hv, e, D // 2 + g * 16)
                cc_a, cc_b = unpack2(s_v, ce, D // 2 + g * 16)
                acc = acc + jnp.maximum(hs_a + cc_a, 0.0) * wk[2 * g]
                acc = acc + jnp.maximum(hs_b + cc_b, 0.0) * wk[2 * g + 1]
            a = jnp.sum(acc)
            alpha = 1.0 / (1.0 + jnp.exp(-a * ones16))
            for g in range(D // 32):
                h_a, h_b = unpack2(hv, e, g * 16)
                r_a, r_b = unpack2(s_v, ce, g * 16)
                ov[e, pl.ds(g * 32, 16)] = h_a * r_a * alpha
                ov[e, pl.ds(g * 32 + 16, 16)] = h_b * r_b * alpha

        pltpu.async_copy(ov, out_hbm.at[pl.ds(base + j * K, K)], sem_o[b])

    issue(jnp.int32(0), 0)

    def chunk_body(i, carry):
        j0 = 2 * i
        j1 = j0 + 1
        issue(j1, 1)
        wait(j0, 0)
        compute(j0, 0)
        issue(j0 + 2, 0)
        wait(j1, 1)
        compute(j1, 1)
        return carry

    lax.fori_loop(0, HALF, chunk_body, jnp.int32(0))
    last = jnp.int32(NCHUNK - 1)
    wait(last, 0)
    compute(last, 0)
    pltpu.make_async_copy(
        out0, out_hbm.at[pl.ds(base + (NCHUNK - 1) * K, K)], sem_o0).wait()
    pltpu.make_async_copy(
        out1, out_hbm.at[pl.ds(base + (NCHUNK - 2) * K, K)], sem_o1).wait()


def kernel(query, q_sub, q_rel, hidden, edges, nodes, rela_embed,
           Ws_attn, Wr_attn, Wqr_attn_W, Wqr_attn_b, W_attn):
    all_ent = hidden.reshape(-1, D)
    blk = 2000
    nsteps = all_ent.shape[0] // blk
    erows = E // D
    eblk = erows // nsteps
    zero_map = lambda i: (0, 0)
    node_table, combo_table = pl.pallas_call(
        _prep_body,
        grid=(nsteps,),
        in_specs=[
            pl.BlockSpec((blk, D), lambda i: (i, 0)),
            pl.BlockSpec((D, D), zero_map),
            pl.BlockSpec((R, D), zero_map),
            pl.BlockSpec((B, D), zero_map),
            pl.BlockSpec((D, D), zero_map),
            pl.BlockSpec((D, D), zero_map),
            pl.BlockSpec((1, D), zero_map),
        ],
        out_specs=[
            pl.BlockSpec((blk, D), lambda i: (i, 0)),
            pl.BlockSpec((R * B, D), zero_map),
        ],
        out_shape=[
            jax.ShapeDtypeStruct((all_ent.shape[0], D), jnp.int32),
            jax.ShapeDtypeStruct((R * B, D), jnp.int32),
        ],
    )(all_ent, Ws_attn, rela_embed, query, Wr_attn, Wqr_attn_W,
      Wqr_attn_b.reshape(1, D))

    sub_idx = edges[:, 1]
    rel_idx = edges[:, 2]
    bat_idx = edges[:, 0]

    mesh = plsc.VectorSubcoreMesh(
        core_axis_name="c", subcore_axis_name="s",
        num_cores=NC, num_subcores=NS)
    sc = functools.partial(
        pl.kernel,
        mesh=mesh,
        compiler_params=pltpu.CompilerParams(needs_layout_passes=False),
        out_type=jax.ShapeDtypeStruct((E, D), jnp.float32),
        scratch_types=[
            pltpu.VMEM((EPW,), jnp.int32),
            pltpu.VMEM((EPW + 16,), jnp.int32),
            pltpu.VMEM((EPW,), jnp.int32),
            pltpu.VMEM((R * B, D), jnp.int32),
            pltpu.VMEM((K, D), jnp.int32),
            pltpu.VMEM((K, D), jnp.int32),
            pltpu.VMEM((K, D), jnp.float32),
            pltpu.VMEM((K, D), jnp.float32),
            pltpu.VMEM((D,), jnp.float32),
            pltpu.SemaphoreType.DMA,
            pltpu.SemaphoreType.DMA,
            pltpu.SemaphoreType.DMA,
            pltpu.SemaphoreType.DMA,
        ],
    )(_edge_body)
    return sc(node_table, combo_table, sub_idx, rel_idx, bat_idx,
              W_attn.reshape(D))


# speedup vs baseline: 1.6253x; 1.0672x over previous
import functools

import jax
import jax.numpy as jnp
from jax import lax
from jax.experimental import pallas as pl
from jax.experimental.pallas import tpu as pltpu
from jax.experimental.pallas import tpu_sc as plsc

B = 8
N = 10000
D = 128
E = 320000
R = 32

NC = 2
NS = 16
NW = NC * NS
EPW = E // NW
K = 80
NCHUNK = EPW // K
HALF = NCHUNK // 2


def _chunk_perms():
    r = lax.broadcasted_iota(jnp.int32, (2 * D, D), 0)
    c = lax.broadcasted_iota(jnp.int32, (2 * D, D), 1)
    feat_a = 32 * (c // 16) + (c % 16)
    pa = (r == feat_a).astype(jnp.float32)
    pb = (r == feat_a + 16).astype(jnp.float32)
    return pa, pb


def _pack_i32(lo_half, hi_half):
    pa, pb = _chunk_perms()
    lo16 = lo_half.astype(jnp.bfloat16)
    hi16 = hi_half.astype(jnp.bfloat16)
    a = jnp.dot(lo16, pa[:D].astype(jnp.bfloat16),
                preferred_element_type=jnp.float32) + \
        jnp.dot(hi16, pa[D:].astype(jnp.bfloat16),
                preferred_element_type=jnp.float32)
    b = jnp.dot(lo16, pb[:D].astype(jnp.bfloat16),
                preferred_element_type=jnp.float32) + \
        jnp.dot(hi16, pb[D:].astype(jnp.bfloat16),
                preferred_element_type=jnp.float32)
    au = lax.bitcast_convert_type(a.astype(jnp.bfloat16), jnp.uint16)
    bu = lax.bitcast_convert_type(b.astype(jnp.bfloat16), jnp.uint16)
    return au.astype(jnp.int32) | (bu.astype(jnp.int32) << 16)


def _prep_body(a_ref, ws_ref, rela_ref, q_ref, wr_ref, wq_ref, b_ref,
               t_ref, s_ref):
    pa, pb = _chunk_perms()
    ws = ws_ref[...]
    ma = (pa[:D] + jnp.dot(ws, pa[D:], preferred_element_type=jnp.float32))
    mb = (pb[:D] + jnp.dot(ws, pb[D:], preferred_element_type=jnp.float32))
    a16 = a_ref[...].astype(jnp.bfloat16)
    va = jnp.dot(a16, ma.astype(jnp.bfloat16), preferred_element_type=jnp.float32)
    vb = jnp.dot(a16, mb.astype(jnp.bfloat16), preferred_element_type=jnp.float32)
    au = lax.bitcast_convert_type(va.astype(jnp.bfloat16), jnp.uint16)
    bu = lax.bitcast_convert_type(vb.astype(jnp.bfloat16), jnp.uint16)
    t_ref[...] = au.astype(jnp.int32) | (bu.astype(jnp.int32) << 16)

    @pl.when(pl.program_id(0) == 0)
    def _():
        rela = rela_ref[...]
        rr = jnp.dot(rela, wr_ref[...], preferred_element_type=jnp.float32)
        qw = jnp.dot(q_ref[...], wq_ref[...], preferred_element_type=jnp.float32)
        qw = qw + b_ref[...]
        rrep = jnp.broadcast_to(rela[:, None, :], (R, B, D)).reshape(R * B, D)
        s2 = (rr[:, None, :] + qw[None, :, :]).reshape(R * B, D)
        s_ref[...] = _pack_i32(rrep, s2)


def _edge_body(t_hbm, s_hbm, sub_hbm, rel_hbm, bat_hbm, w_hbm, out_hbm,
               sub_all, c_all, bat_all, s_v, hrows0, hrows1, out0, out1,
               w_v, sem_t0, sem_t1, sem_o0, sem_o1):
    wid = lax.axis_index("s") * NC + lax.axis_index("c")
    base = wid * EPW
    pltpu.sync_copy(w_hbm, w_v)
    pltpu.sync_copy(s_hbm, s_v)
    pltpu.sync_copy(sub_hbm.at[pl.ds(base, EPW)], sub_all)
    pltpu.sync_copy(rel_hbm.at[pl.ds(base, EPW)], c_all.at[pl.ds(0, EPW)])
    pltpu.sync_copy(bat_hbm.at[pl.ds(base, EPW)], bat_all)

    @plsc.parallel_loop(0, EPW, 16, unroll=4)
    def idx_body(i):
        sl = pl.ds(i, 16)
        c_all[sl] = c_all[sl] * 8 + bat_all[sl]

    ones16 = jnp.ones((16,), jnp.float32)
    wk = [w_v[pl.ds(k * 16, 16)] for k in range(D // 16)]
    hrows = (hrows0, hrows1)
    outs = (out0, out1)
    sem_t = (sem_t0, sem_t1)
    sem_o = (sem_o0, sem_o1)

    def issue(j, b):
        pltpu.async_copy(t_hbm.at[sub_all.at[pl.ds(j * K, K)]], hrows[b], sem_t[b])

    def wait(j, b):
        pltpu.make_async_copy(
            t_hbm.at[sub_all.at[pl.ds(j * K, K)]], hrows[b], sem_t[b]).wait()

    def compute(j, b):
        hv, ov = hrows[b], outs[b]
        ebase = j * K

        @pl.when(j >= 2)
        def _():
            pltpu.make_async_copy(
                ov, out_hbm.at[pl.ds(base + (j - 2) * K, K)], sem_o[b]).wait()

        def unpack2(ref, row, word_ofs):
            q = ref[row, pl.ds(word_ofs, 16)]
            bc = plsc.bitcast(q, jnp.bfloat16)
            return plsc.unpack(bc, format=plsc.PackFormat.INTERLEAVED)

        @plsc.parallel_loop(0, K, 1, unroll=4)
        def edge_body(e):
            ce = c_all[pl.ds(ebase + e, 16)][0]
            acc = jnp.zeros((16,), jnp.float32)
            for g in range(D // 32):
                hs_a, hs_b = unpack2(hv, e, D // 2 + g * 16)
                cc_a, cc_b = unpack2(s_v, ce, D // 2 + g * 16)
                acc = acc + jnp.maximum(hs_a + cc_a, 0.0) * wk[2 * g]
                acc = acc + jnp.maximum(hs_b + cc_b, 0.0) * wk[2 * g + 1]
            a = jnp.sum(acc)
            alpha = 1.0 / (1.0 + jnp.exp(-a * ones16))
            for g in range(D // 32):
                h_a, h_b = unpack2(hv, e, g * 16)
                r_a, r_b = unpack2(s_v, ce, g * 16)
                ov[e, pl.ds(g * 32, 16)] = h_a * r_a * alpha
                ov[e, pl.ds(g * 32 + 16, 16)] = h_b * r_b * alpha

        pltpu.async_copy(ov, out_hbm.at[pl.ds(base + j * K, K)], sem_o[b])

    issue(jnp.int32(0), 0)

    def chunk_body(i, carry):
        j0 = 2 * i
        j1 = j0 + 1
        issue(j1, 1)
        wait(j0, 0)
        compute(j0, 0)
        issue(j0 + 2, 0)
        wait(j1, 1)
        compute(j1, 1)
        return carry

    lax.fori_loop(0, HALF, chunk_body, jnp.int32(0))
    last = jnp.int32(NCHUNK - 1)
    wait(last, 0)
    compute(last, 0)
    pltpu.make_async_copy(
        out0, out_hbm.at[pl.ds(base + (NCHUNK - 1) * K, K)], sem_o0).wait()
    pltpu.make_async_copy(
        out1, out_hbm.at[pl.ds(base + (NCHUNK - 2) * K, K)], sem_o1).wait()


def kernel(query, q_sub, q_rel, hidden, edges, nodes, rela_embed,
           Ws_attn, Wr_attn, Wqr_attn_W, Wqr_attn_b, W_attn):
    all_ent = hidden.reshape(-1, D)
    blk = 4000
    nsteps = all_ent.shape[0] // blk
    erows = E // D
    eblk = erows // nsteps
    zero_map = lambda i: (0, 0)
    node_table, combo_table = pl.pallas_call(
        _prep_body,
        grid=(nsteps,),
        in_specs=[
            pl.BlockSpec((blk, D), lambda i: (i, 0)),
            pl.BlockSpec((D, D), zero_map),
            pl.BlockSpec((R, D), zero_map),
            pl.BlockSpec((B, D), zero_map),
            pl.BlockSpec((D, D), zero_map),
            pl.BlockSpec((D, D), zero_map),
            pl.BlockSpec((1, D), zero_map),
        ],
        out_specs=[
            pl.BlockSpec((blk, D), lambda i: (i, 0)),
            pl.BlockSpec((R * B, D), zero_map),
        ],
        out_shape=[
            jax.ShapeDtypeStruct((all_ent.shape[0], D), jnp.int32),
            jax.ShapeDtypeStruct((R * B, D), jnp.int32),
        ],
    )(all_ent, Ws_attn, rela_embed, query, Wr_attn, Wqr_attn_W,
      Wqr_attn_b.reshape(1, D))

    sub_idx = edges[:, 1]
    rel_idx = edges[:, 2]
    bat_idx = edges[:, 0]

    mesh = plsc.VectorSubcoreMesh(
        core_axis_name="c", subcore_axis_name="s",
        num_cores=NC, num_subcores=NS)
    sc = functools.partial(
        pl.kernel,
        mesh=mesh,
        compiler_params=pltpu.CompilerParams(needs_layout_passes=False),
        out_type=jax.ShapeDtypeStruct((E, D), jnp.float32),
        scratch_types=[
            pltpu.VMEM((EPW,), jnp.int32),
            pltpu.VMEM((EPW + 16,), jnp.int32),
            pltpu.VMEM((EPW,), jnp.int32),
            pltpu.VMEM((R * B, D), jnp.int32),
            pltpu.VMEM((K, D), jnp.int32),
            pltpu.VMEM((K, D), jnp.int32),
            pltpu.VMEM((K, D), jnp.float32),
            pltpu.VMEM((K, D), jnp.float32),
            pltpu.VMEM((D,), jnp.float32),
            pltpu.SemaphoreType.DMA,
            pltpu.SemaphoreType.DMA,
            pltpu.SemaphoreType.DMA,
            pltpu.SemaphoreType.DMA,
        ],
    )(_edge_body)
    return sc(node_table, combo_table, sub_idx, rel_idx, bat_idx,
              W_attn.reshape(D))


# speedup vs baseline: 1.6377x; 1.0076x over previous
import functools

import jax
import jax.numpy as jnp
from jax import lax
from jax.experimental import pallas as pl
from jax.experimental.pallas import tpu as pltpu
from jax.experimental.pallas import tpu_sc as plsc

B = 8
N = 10000
D = 128
E = 320000
R = 32

NC = 2
NS = 16
NW = NC * NS
EPW = E // NW
K = 80
NCHUNK = EPW // K
HALF = NCHUNK // 2


def _chunk_perms():
    r = lax.broadcasted_iota(jnp.int32, (2 * D, D), 0)
    c = lax.broadcasted_iota(jnp.int32, (2 * D, D), 1)
    feat_a = 32 * (c // 16) + (c % 16)
    pa = (r == feat_a).astype(jnp.float32)
    pb = (r == feat_a + 16).astype(jnp.float32)
    return pa, pb


def _pack_i32(lo_half, hi_half):
    pa, pb = _chunk_perms()
    lo16 = lo_half.astype(jnp.bfloat16)
    hi16 = hi_half.astype(jnp.bfloat16)
    a = jnp.dot(lo16, pa[:D].astype(jnp.bfloat16),
                preferred_element_type=jnp.float32) + \
        jnp.dot(hi16, pa[D:].astype(jnp.bfloat16),
                preferred_element_type=jnp.float32)
    b = jnp.dot(lo16, pb[:D].astype(jnp.bfloat16),
                preferred_element_type=jnp.float32) + \
        jnp.dot(hi16, pb[D:].astype(jnp.bfloat16),
                preferred_element_type=jnp.float32)
    au = lax.bitcast_convert_type(a.astype(jnp.bfloat16), jnp.uint16)
    bu = lax.bitcast_convert_type(b.astype(jnp.bfloat16), jnp.uint16)
    return au.astype(jnp.int32) | (bu.astype(jnp.int32) << 16)


def _prep_body(a_ref, ws_ref, rela_ref, q_ref, wr_ref, wq_ref, b_ref,
               t_ref, s_ref):
    pa, pb = _chunk_perms()
    ws = ws_ref[...]
    ma = (pa[:D] + jnp.dot(ws, pa[D:], preferred_element_type=jnp.float32))
    mb = (pb[:D] + jnp.dot(ws, pb[D:], preferred_element_type=jnp.float32))
    a16 = a_ref[...].astype(jnp.bfloat16)
    va = jnp.dot(a16, ma.astype(jnp.bfloat16), preferred_element_type=jnp.float32)
    vb = jnp.dot(a16, mb.astype(jnp.bfloat16), preferred_element_type=jnp.float32)
    au = lax.bitcast_convert_type(va.astype(jnp.bfloat16), jnp.uint16)
    bu = lax.bitcast_convert_type(vb.astype(jnp.bfloat16), jnp.uint16)
    t_ref[...] = au.astype(jnp.int32) | (bu.astype(jnp.int32) << 16)

    @pl.when(pl.program_id(0) == 0)
    def _():
        rela = rela_ref[...]
        rr = jnp.dot(rela, wr_ref[...], preferred_element_type=jnp.float32)
        qw = jnp.dot(q_ref[...], wq_ref[...], preferred_element_type=jnp.float32)
        qw = qw + b_ref[...]
        rrep = jnp.broadcast_to(rela[:, None, :], (R, B, D)).reshape(R * B, D)
        s2 = (rr[:, None, :] + qw[None, :, :]).reshape(R * B, D)
        s_ref[...] = _pack_i32(rrep, s2)


def _edge_body(t_hbm, s_hbm, sub_hbm, rel_hbm, bat_hbm, w_hbm, out_hbm,
               sub_all, c_all, bat_all, s_v, hrows0, hrows1, out0, out1,
               w_v, sem_t0, sem_t1, sem_o0, sem_o1):
    wid = lax.axis_index("s") * NC + lax.axis_index("c")
    base = wid * EPW
    pltpu.sync_copy(w_hbm, w_v)
    pltpu.sync_copy(s_hbm, s_v)
    pltpu.sync_copy(sub_hbm.at[pl.ds(base, EPW)], sub_all)
    pltpu.sync_copy(rel_hbm.at[pl.ds(base, EPW)], c_all.at[pl.ds(0, EPW)])
    pltpu.sync_copy(bat_hbm.at[pl.ds(base, EPW)], bat_all)

    @plsc.parallel_loop(0, EPW, 16, unroll=4)
    def idx_body(i):
        sl = pl.ds(i, 16)
        c_all[sl] = c_all[sl] * 8 + bat_all[sl]

    ones16 = jnp.ones((16,), jnp.float32)
    wk = [w_v[pl.ds(k * 16, 16)] for k in range(D // 16)]
    hrows = (hrows0, hrows1)
    outs = (out0, out1)
    sem_t = (sem_t0, sem_t1)
    sem_o = (sem_o0, sem_o1)

    def issue(j, b):
        pltpu.async_copy(t_hbm.at[sub_all.at[pl.ds(j * K, K)]], hrows[b], sem_t[b])

    def wait(j, b):
        pltpu.make_async_copy(
            t_hbm.at[sub_all.at[pl.ds(j * K, K)]], hrows[b], sem_t[b]).wait()

    def compute(j, b):
        hv, ov = hrows[b], outs[b]
        ebase = j * K

        @pl.when(j >= 2)
        def _():
            pltpu.make_async_copy(
                ov, out_hbm.at[pl.ds(base + (j - 2) * K, K)], sem_o[b]).wait()

        def unpack2(ref, row, word_ofs):
            q = ref[row, pl.ds(word_ofs, 16)]
            bc = plsc.bitcast(q, jnp.bfloat16)
            return plsc.unpack(bc, format=plsc.PackFormat.INTERLEAVED)

        @plsc.parallel_loop(0, K, 1, unroll=4)
        def edge_body(e):
            ce = c_all[pl.ds(ebase + e, 16)][0]
            acc = jnp.zeros((16,), jnp.float32)
            for g in range(D // 32):
                hs_a, hs_b = unpack2(hv, e, D // 2 + g * 16)
                cc_a, cc_b = unpack2(s_v, ce, D // 2 + g * 16)
                acc = acc + jnp.maximum(hs_a + cc_a, 0.0) * wk[2 * g]
                acc = acc + jnp.maximum(hs_b + cc_b, 0.0) * wk[2 * g + 1]
            a = jnp.sum(acc)
            alpha = 1.0 / (1.0 + jnp.exp(-a * ones16))
            for g in range(D // 32):
                h_a, h_b = unpack2(hv, e, g * 16)
                r_a, r_b = unpack2(s_v, ce, g * 16)
                ov[e, pl.ds(g * 32, 16)] = h_a * r_a * alpha
                ov[e, pl.ds(g * 32 + 16, 16)] = h_b * r_b * alpha

        pltpu.async_copy(ov, out_hbm.at[pl.ds(base + j * K, K)], sem_o[b])

    issue(jnp.int32(0), 0)

    def chunk_body(i, carry):
        j0 = 2 * i
        j1 = j0 + 1
        issue(j1, 1)
        wait(j0, 0)
        compute(j0, 0)
        issue(j0 + 2, 0)
        wait(j1, 1)
        compute(j1, 1)
        return carry

    lax.fori_loop(0, HALF, chunk_body, jnp.int32(0))
    last = jnp.int32(NCHUNK - 1)
    wait(last, 0)
    compute(last, 0)
    pltpu.make_async_copy(
        out0, out_hbm.at[pl.ds(base + (NCHUNK - 1) * K, K)], sem_o0).wait()
    pltpu.make_async_copy(
        out1, out_hbm.at[pl.ds(base + (NCHUNK - 2) * K, K)], sem_o1).wait()


def kernel(query, q_sub, q_rel, hidden, edges, nodes, rela_embed,
           Ws_attn, Wr_attn, Wqr_attn_W, Wqr_attn_b, W_attn):
    all_ent = hidden.reshape(-1, D)
    blk = 10000
    nsteps = all_ent.shape[0] // blk
    erows = E // D
    eblk = erows // nsteps
    zero_map = lambda i: (0, 0)
    node_table, combo_table = pl.pallas_call(
        _prep_body,
        grid=(nsteps,),
        in_specs=[
            pl.BlockSpec((blk, D), lambda i: (i, 0)),
            pl.BlockSpec((D, D), zero_map),
            pl.BlockSpec((R, D), zero_map),
            pl.BlockSpec((B, D), zero_map),
            pl.BlockSpec((D, D), zero_map),
            pl.BlockSpec((D, D), zero_map),
            pl.BlockSpec((1, D), zero_map),
        ],
        out_specs=[
            pl.BlockSpec((blk, D), lambda i: (i, 0)),
            pl.BlockSpec((R * B, D), zero_map),
        ],
        out_shape=[
            jax.ShapeDtypeStruct((all_ent.shape[0], D), jnp.int32),
            jax.ShapeDtypeStruct((R * B, D), jnp.int32),
        ],
    )(all_ent, Ws_attn, rela_embed, query, Wr_attn, Wqr_attn_W,
      Wqr_attn_b.reshape(1, D))

    sub_idx = edges[:, 1]
    rel_idx = edges[:, 2]
    bat_idx = edges[:, 0]

    mesh = plsc.VectorSubcoreMesh(
        core_axis_name="c", subcore_axis_name="s",
        num_cores=NC, num_subcores=NS)
    sc = functools.partial(
        pl.kernel,
        mesh=mesh,
        compiler_params=pltpu.CompilerParams(needs_layout_passes=False),
        out_type=jax.ShapeDtypeStruct((E, D), jnp.float32),
        scratch_types=[
            pltpu.VMEM((EPW,), jnp.int32),
            pltpu.VMEM((EPW + 16,), jnp.int32),
            pltpu.VMEM((EPW,), jnp.int32),
            pltpu.VMEM((R * B, D), jnp.int32),
            pltpu.VMEM((K, D), jnp.int32),
            pltpu.VMEM((K, D), jnp.int32),
            pltpu.VMEM((K, D), jnp.float32),
            pltpu.VMEM((K, D), jnp.float32),
            pltpu.VMEM((D,), jnp.float32),
            pltpu.SemaphoreType.DMA,
            pltpu.SemaphoreType.DMA,
            pltpu.SemaphoreType.DMA,
            pltpu.SemaphoreType.DMA,
        ],
    )(_edge_body)
    return sc(node_table, combo_table, sub_idx, rel_idx, bat_idx,
              W_attn.reshape(D))


# speedup vs baseline: 1.6412x; 1.0022x over previous
import functools

import jax
import jax.numpy as jnp
from jax import lax
from jax.experimental import pallas as pl
from jax.experimental.pallas import tpu as pltpu
from jax.experimental.pallas import tpu_sc as plsc

B = 8
N = 10000
D = 128
E = 320000
R = 32

NC = 2
NS = 16
NW = NC * NS
EPW = E // NW
K = 80
NCHUNK = EPW // K
HALF = NCHUNK // 2


def _chunk_perms():
    r = lax.broadcasted_iota(jnp.int32, (2 * D, D), 0)
    c = lax.broadcasted_iota(jnp.int32, (2 * D, D), 1)
    feat_a = 32 * (c // 16) + (c % 16)
    pa = (r == feat_a).astype(jnp.float32)
    pb = (r == feat_a + 16).astype(jnp.float32)
    return pa, pb


def _pack_i32(lo_half, hi_half):
    pa, pb = _chunk_perms()
    lo16 = lo_half.astype(jnp.bfloat16)
    hi16 = hi_half.astype(jnp.bfloat16)
    a = jnp.dot(lo16, pa[:D].astype(jnp.bfloat16),
                preferred_element_type=jnp.float32) + \
        jnp.dot(hi16, pa[D:].astype(jnp.bfloat16),
                preferred_element_type=jnp.float32)
    b = jnp.dot(lo16, pb[:D].astype(jnp.bfloat16),
                preferred_element_type=jnp.float32) + \
        jnp.dot(hi16, pb[D:].astype(jnp.bfloat16),
                preferred_element_type=jnp.float32)
    au = lax.bitcast_convert_type(a.astype(jnp.bfloat16), jnp.uint16)
    bu = lax.bitcast_convert_type(b.astype(jnp.bfloat16), jnp.uint16)
    return au.astype(jnp.int32) | (bu.astype(jnp.int32) << 16)


def _prep_body(a_ref, ws_ref, rela_ref, q_ref, wr_ref, wq_ref, b_ref,
               t_ref, s_ref):
    pa, pb = _chunk_perms()
    ws = ws_ref[...]
    ma = (pa[:D] + jnp.dot(ws, pa[D:], preferred_element_type=jnp.float32))
    mb = (pb[:D] + jnp.dot(ws, pb[D:], preferred_element_type=jnp.float32))
    a16 = a_ref[...].astype(jnp.bfloat16)
    va = jnp.dot(a16, ma.astype(jnp.bfloat16), preferred_element_type=jnp.float32)
    vb = jnp.dot(a16, mb.astype(jnp.bfloat16), preferred_element_type=jnp.float32)
    au = lax.bitcast_convert_type(va.astype(jnp.bfloat16), jnp.uint16)
    bu = lax.bitcast_convert_type(vb.astype(jnp.bfloat16), jnp.uint16)
    t_ref[...] = au.astype(jnp.int32) | (bu.astype(jnp.int32) << 16)

    @pl.when(pl.program_id(0) == 0)
    def _():
        rela = rela_ref[...]
        rr = jnp.dot(rela, wr_ref[...], preferred_element_type=jnp.float32)
        qw = jnp.dot(q_ref[...], wq_ref[...], preferred_element_type=jnp.float32)
        qw = qw + b_ref[...]
        rrep = jnp.broadcast_to(rela[:, None, :], (R, B, D)).reshape(R * B, D)
        s2 = (rr[:, None, :] + qw[None, :, :]).reshape(R * B, D)
        s_ref[...] = _pack_i32(rrep, s2)


def _edge_body(t_hbm, s_hbm, sub_hbm, rel_hbm, bat_hbm, w_hbm, out_hbm,
               sub_all, c_all, bat_all, s_v, hrows0, hrows1, out0, out1,
               w_v, sem_t0, sem_t1, sem_o0, sem_o1):
    wid = lax.axis_index("s") * NC + lax.axis_index("c")
    base = wid * EPW
    pltpu.sync_copy(w_hbm, w_v)
    pltpu.sync_copy(s_hbm, s_v)
    pltpu.sync_copy(sub_hbm.at[pl.ds(base, EPW)], sub_all)
    pltpu.sync_copy(rel_hbm.at[pl.ds(base, EPW)], c_all.at[pl.ds(0, EPW)])
    pltpu.sync_copy(bat_hbm.at[pl.ds(base, EPW)], bat_all)

    @plsc.parallel_loop(0, EPW, 16, unroll=4)
    def idx_body(i):
        sl = pl.ds(i, 16)
        c_all[sl] = c_all[sl] * 8 + bat_all[sl]

    ones16 = jnp.ones((16,), jnp.float32)
    wk = [w_v[pl.ds(k * 16, 16)] for k in range(D // 16)]
    hrows = (hrows0, hrows1)
    outs = (out0, out1)
    sem_t = (sem_t0, sem_t1)
    sem_o = (sem_o0, sem_o1)

    def issue(j, b):
        pltpu.async_copy(t_hbm.at[sub_all.at[pl.ds(j * K, K)]], hrows[b], sem_t[b])

    def wait(j, b):
        pltpu.make_async_copy(
            t_hbm.at[sub_all.at[pl.ds(j * K, K)]], hrows[b], sem_t[b]).wait()

    def compute(j, b):
        hv, ov = hrows[b], outs[b]
        ebase = j * K

        @pl.when(j >= 2)
        def _():
            pltpu.make_async_copy(
                ov, out_hbm.at[pl.ds(base + (j - 2) * K, K)], sem_o[b]).wait()

        def unpack2(ref, row, word_ofs):
            q = ref[row, pl.ds(word_ofs, 16)]
            bc = plsc.bitcast(q, jnp.bfloat16)
            return plsc.unpack(bc, format=plsc.PackFormat.INTERLEAVED)

        @plsc.parallel_loop(0, K, 1, unroll=4)
        def edge_body(e):
            ce = c_all[pl.ds(ebase + e, 16)][0]
            acc = jnp.zeros((16,), jnp.float32)
            for g in range(D // 32):
                hs_a, hs_b = unpack2(hv, e, D // 2 + g * 16)
                cc_a, cc_b = unpack2(s_v, ce, D // 2 + g * 16)
                acc = acc + jnp.maximum(hs_a + cc_a, 0.0) * wk[2 * g]
                acc = acc + jnp.maximum(hs_b + cc_b, 0.0) * wk[2 * g + 1]
            a = jnp.sum(acc)
            alpha = 1.0 / (1.0 + jnp.exp(-a * ones16))
            for g in range(D // 32):
                h_a, h_b = unpack2(hv, e, g * 16)
                r_a, r_b = unpack2(s_v, ce, g * 16)
                ov[e, pl.ds(g * 32, 16)] = h_a * r_a * alpha
                ov[e, pl.ds(g * 32 + 16, 16)] = h_b * r_b * alpha

        pltpu.async_copy(ov, out_hbm.at[pl.ds(base + j * K, K)], sem_o[b])

    issue(jnp.int32(0), 0)

    def chunk_body(i, carry):
        j0 = 2 * i
        j1 = j0 + 1
        issue(j1, 1)
        wait(j0, 0)
        compute(j0, 0)
        issue(j0 + 2, 0)
        wait(j1, 1)
        compute(j1, 1)
        return carry

    lax.fori_loop(0, HALF, chunk_body, jnp.int32(0))
    last = jnp.int32(NCHUNK - 1)
    wait(last, 0)
    compute(last, 0)
    pltpu.make_async_copy(
        out0, out_hbm.at[pl.ds(base + (NCHUNK - 1) * K, K)], sem_o0).wait()
    pltpu.make_async_copy(
        out1, out_hbm.at[pl.ds(base + (NCHUNK - 2) * K, K)], sem_o1).wait()


def kernel(query, q_sub, q_rel, hidden, edges, nodes, rela_embed,
           Ws_attn, Wr_attn, Wqr_attn_W, Wqr_attn_b, W_attn):
    all_ent = hidden.reshape(-1, D)
    blk = 10000
    nsteps = all_ent.shape[0] // blk
    zero_map = lambda i: (0, 0)
    node_table, combo_table = pl.pallas_call(
        _prep_body,
        grid=(nsteps,),
        in_specs=[
            pl.BlockSpec((blk, D), lambda i: (i, 0)),
            pl.BlockSpec((D, D), zero_map),
            pl.BlockSpec((R, D), zero_map),
            pl.BlockSpec((B, D), zero_map),
            pl.BlockSpec((D, D), zero_map),
            pl.BlockSpec((D, D), zero_map),
            pl.BlockSpec((1, D), zero_map),
        ],
        out_specs=[
            pl.BlockSpec((blk, D), lambda i: (i, 0)),
            pl.BlockSpec((R * B, D), zero_map),
        ],
        out_shape=[
            jax.ShapeDtypeStruct((all_ent.shape[0], D), jnp.int32),
            jax.ShapeDtypeStruct((R * B, D), jnp.int32),
        ],
    )(all_ent, Ws_attn, rela_embed, query, Wr_attn, Wqr_attn_W,
      Wqr_attn_b.reshape(1, D))

    sub_idx = edges[:, 1]
    rel_idx = edges[:, 2]
    bat_idx = edges[:, 0]

    mesh = plsc.VectorSubcoreMesh(
        core_axis_name="c", subcore_axis_name="s",
        num_cores=NC, num_subcores=NS)
    sc = functools.partial(
        pl.kernel,
        mesh=mesh,
        compiler_params=pltpu.CompilerParams(needs_layout_passes=False),
        out_type=jax.ShapeDtypeStruct((E, D), jnp.float32),
        scratch_types=[
            pltpu.VMEM((EPW,), jnp.int32),
            pltpu.VMEM((EPW + 16,), jnp.int32),
            pltpu.VMEM((EPW,), jnp.int32),
            pltpu.VMEM((R * B, D), jnp.int32),
            pltpu.VMEM((K, D), jnp.int32),
            pltpu.VMEM((K, D), jnp.int32),
            pltpu.VMEM((K, D), jnp.float32),
            pltpu.VMEM((K, D), jnp.float32),
            pltpu.VMEM((D,), jnp.float32),
            pltpu.SemaphoreType.DMA,
            pltpu.SemaphoreType.DMA,
            pltpu.SemaphoreType.DMA,
            pltpu.SemaphoreType.DMA,
        ],
    )(_edge_body)
    return sc(node_table, combo_table, sub_idx, rel_idx, bat_idx,
              W_attn.reshape(D))
